# baseline (speedup 1.0000x reference)
import functools

import jax
import jax.numpy as jnp
from jax import lax
from jax.experimental import pallas as pl
from jax.experimental.pallas import tpu as pltpu
from jax.experimental.pallas import tpu_sc as plsc

_B, _S, _D = 4096, 26, 64
_N = _B * _S
_NW = 32
_CH = 128
_NSLAB = _S


def _gather(xk, table):
    mesh = plsc.VectorSubcoreMesh(core_axis_name="c", subcore_axis_name="s")

    @functools.partial(
        pl.kernel,
        mesh=mesh,
        out_type=jax.ShapeDtypeStruct((_S, 8, _NW, 8, 128), jnp.float32),
        compiler_params=pltpu.CompilerParams(
            use_tc_tiling_on_sc=False, needs_layout_passes=False),
        scratch_types=[
            pltpu.VMEM((2, _CH), jnp.int32),
            pltpu.VMEM((2, _CH, _D), jnp.float32),
            pltpu.VMEM((2, 8, 8, 128), jnp.float32),
            pltpu.SemaphoreType.DMA,
            pltpu.SemaphoreType.DMA,
            pltpu.SemaphoreType.DMA,
            pltpu.SemaphoreType.DMA,
            pltpu.SemaphoreType.DMA,
            pltpu.SemaphoreType.DMA,
        ],
    )
    def body(xk_hbm, table_hbm, out_hbm, kv, rows_v, slab_v,
             k0, k1, g0, g1, s0, s1):
        ksems = (k0, k1)
        gsems = (g0, g1)
        ssems = (s0, s1)
        w = lax.axis_index("s") * 2 + lax.axis_index("c")
        lane16 = lax.iota(jnp.int32, 16)

        def key_off(s):
            return (s * _NW + w) * _CH

        pltpu.sync_copy(xk_hbm.at[pl.ds(key_off(0), _CH)], kv.at[0])
        pltpu.async_copy(table_hbm.at[kv.at[0]], rows_v.at[0], gsems[0])
        pltpu.async_copy(xk_hbm.at[pl.ds(key_off(1), _CH)], kv.at[1], ksems[1])

        def pair(jj, carry):
            for b in range(2):
                nb = 1 - b
                s = jj * 2 + b

                @pl.when(s + 1 < _NSLAB)
                def _():
                    pltpu.make_async_copy(
                        xk_hbm.at[pl.ds(key_off(s + 1), _CH)],
                        kv.at[nb], ksems[nb]).wait()

                pltpu.make_async_copy(
                    table_hbm.at[kv.at[b]], rows_v.at[b], gsems[b]).wait()

                @pl.when(s + 1 < _NSLAB)
                def _():
                    pltpu.async_copy(
                        table_hbm.at[kv.at[nb]], rows_v.at[nb], gsems[nb])

                @pl.when(s + 2 < _NSLAB)
                def _():
                    pltpu.async_copy(
                        xk_hbm.at[pl.ds(key_off(s + 2), _CH)],
                        kv.at[b], ksems[b])

                @pl.when(s >= 2)
                def _():
                    pltpu.make_async_copy(
                        slab_v.at[b], out_hbm.at[s - 2, :, w], ssems[b]).wait()

                def col(dt, carry2):
                    for sl in range(8):
                        d = dt * 8 + sl
                        for j0 in range(8):
                            vals = plsc.load_gather(
                                rows_v.at[b],
                                [j0 * 16 + lane16,
                                 jnp.full((16,), d, jnp.int32)])
                            slab_v[b, dt, sl, pl.ds(j0 * 16, 16)] = vals
                    return carry2

                lax.fori_loop(0, 8, col, 0)
                pltpu.async_copy(slab_v.at[b], out_hbm.at[s, :, w], ssems[b])
            return carry

        lax.fori_loop(0, _NSLAB // 2, pair, 0)
        for b in range(2):
            s_last = _NSLAB - 2 + b
            pltpu.make_async_copy(
                slab_v.at[b], out_hbm.at[s_last, :, w], ssems[b]).wait()

    return body(xk, table)


def kernel(x, table):
    xk = (x.astype(jnp.int32)
          .reshape(_NW, _CH, _S)
          .transpose(2, 0, 1)
          .reshape(_N))
    out = _gather(xk, table)
    return out.transpose(2, 4, 0, 1, 3).reshape(_B, _S, _D)


# speedup vs baseline: 1.1388x; 1.1388x over previous
import functools

import jax
import jax.numpy as jnp
from jax import lax
from jax.experimental import pallas as pl
from jax.experimental.pallas import tpu as pltpu
from jax.experimental.pallas import tpu_sc as plsc

_B, _S, _D = 4096, 26, 64
_NW = 32
_CH = 128
_NSLAB = _S


def _gather(xk, table):
    mesh = plsc.VectorSubcoreMesh(core_axis_name="c", subcore_axis_name="s")

    @functools.partial(
        pl.kernel,
        mesh=mesh,
        out_type=jax.ShapeDtypeStruct((_S, 8, _NW, 1024), jnp.float32),
        compiler_params=pltpu.CompilerParams(
            use_tc_tiling_on_sc=False, needs_layout_passes=False),
        scratch_types=[
            pltpu.VMEM((_S, _CH), jnp.int32),
            pltpu.VMEM((2, _CH, _D), jnp.float32),
            pltpu.VMEM((2, 8 * 1024), jnp.float32),
            pltpu.SemaphoreType.DMA,
            pltpu.SemaphoreType.DMA,
            pltpu.SemaphoreType.DMA,
            pltpu.SemaphoreType.DMA,
        ],
    )
    def body(xk_hbm, table_hbm, out_hbm, kv, rows_v, slab_v, g0, g1, s0, s1):
        gsems = (g0, g1)
        ssems = (s0, s1)
        w = lax.axis_index("s") * 2 + lax.axis_index("c")
        lane16 = lax.iota(jnp.int32, 16)
        dk = [(k * 16 + lane16) * 128 for k in range(4)]

        pltpu.sync_copy(xk_hbm.at[:, w], kv)
        pltpu.async_copy(table_hbm.at[kv.at[0]], rows_v.at[0], gsems[0])

        def out_slab_copy(s, b):
            return [
                pltpu.make_async_copy(
                    slab_v.at[b, pl.ds(dt * 1024, 1024)],
                    out_hbm.at[s, dt, w], ssems[b])
                for dt in range(8)
            ]

        def pair(jj, carry):
            for b in range(2):
                nb = 1 - b
                s = jj * 2 + b

                @pl.when(s + 1 < _NSLAB)
                def _():
                    pltpu.async_copy(
                        table_hbm.at[kv.at[s + 1]], rows_v.at[nb], gsems[nb])

                pltpu.make_async_copy(
                    table_hbm.at[kv.at[s]], rows_v.at[b], gsems[b]).wait()

                @pl.when(s >= 2)
                def _():
                    for c in out_slab_copy(s - 2, b):
                        c.wait()

                for j in range(_CH):
                    for k in range(4):
                        vals = rows_v[b, j, pl.ds(k * 16, 16)]
                        plsc.store_scatter(slab_v.at[b], [dk[k] + j], vals)

                for c in out_slab_copy(s, b):
                    c.start()
            return carry

        lax.fori_loop(0, _NSLAB // 2, pair, 0)
        for b in range(2):
            for c in out_slab_copy(_NSLAB - 2 + b, b):
                c.wait()

    return body(xk, table)


def kernel(x, table):
    xk = x.astype(jnp.int32).reshape(_NW, _CH, _S).transpose(2, 0, 1)
    out = _gather(xk, table)
    return (out.reshape(_S, 8, _NW, 8, 128)
            .transpose(2, 4, 0, 1, 3)
            .reshape(_B, _S, _D))


# speedup vs baseline: 1.5977x; 1.4030x over previous
import functools

import jax
import jax.numpy as jnp
from jax import lax
from jax.experimental import pallas as pl
from jax.experimental.pallas import tpu as pltpu
from jax.experimental.pallas import tpu_sc as plsc

_B, _S, _D = 4096, 26, 64
_NW = 32
_CH = 128
_NSLAB = _S


def _gather(xk, table):
    mesh = plsc.VectorSubcoreMesh(core_axis_name="c", subcore_axis_name="s")

    @functools.partial(
        pl.kernel,
        mesh=mesh,
        out_type=jax.ShapeDtypeStruct((_S, 8, _NW, 8, 128), jnp.float32),
        compiler_params=pltpu.CompilerParams(
            use_tc_tiling_on_sc=False, needs_layout_passes=False),
        scratch_types=[
            pltpu.VMEM((_S, _CH), jnp.int32),
            pltpu.VMEM((2, _CH, _D), jnp.float32),
            pltpu.VMEM((2, _D, 129), jnp.float32),
            pltpu.SemaphoreType.DMA,
            pltpu.SemaphoreType.DMA,
            pltpu.SemaphoreType.DMA,
            pltpu.SemaphoreType.DMA,
        ],
    )
    def body(xk_hbm, table_hbm, out_hbm, kv, rows_v, slab_v, g0, g1, s0, s1):
        gsems = (g0, g1)
        ssems = (s0, s1)
        w = lax.axis_index("s") * 2 + lax.axis_index("c")
        lane16 = lax.iota(jnp.int32, 16)
        dk = [k * 16 + lane16 for k in range(4)]

        pltpu.sync_copy(xk_hbm.at[:, w], kv)
        pltpu.async_copy(table_hbm.at[kv.at[0]], rows_v.at[0], gsems[0])

        def out_slab_copy(s, b):
            return [
                pltpu.make_async_copy(
                    slab_v.at[b, pl.ds(dt * 8, 8), pl.ds(0, 128)],
                    out_hbm.at[s, dt, w], ssems[b])
                for dt in range(8)
            ]

        def pair(jj, carry):
            for b in range(2):
                nb = 1 - b
                s = jj * 2 + b

                @pl.when(s + 1 < _NSLAB)
                def _():
                    pltpu.async_copy(
                        table_hbm.at[kv.at[s + 1]], rows_v.at[nb], gsems[nb])

                pltpu.make_async_copy(
                    table_hbm.at[kv.at[s]], rows_v.at[b], gsems[b]).wait()

                @pl.when(s >= 2)
                def _():
                    for c in out_slab_copy(s - 2, b):
                        c.wait()

                for j in range(_CH):
                    jv = jnp.full((16,), j, jnp.int32)
                    for k in range(4):
                        vals = rows_v[b, j, pl.ds(k * 16, 16)]
                        plsc.store_scatter(slab_v.at[b], [dk[k], jv], vals)

                for c in out_slab_copy(s, b):
                    c.start()
            return carry

        lax.fori_loop(0, _NSLAB // 2, pair, 0)
        for b in range(2):
            for c in out_slab_copy(_NSLAB - 2 + b, b):
                c.wait()

    return body(xk, table)


def kernel(x, table):
    xk = x.astype(jnp.int32).reshape(_NW, _CH, _S).transpose(2, 0, 1)
    out = _gather(xk, table)
    return out.transpose(2, 4, 0, 1, 3).reshape(_B, _S, _D)
